# Initial kernel scaffold; baseline (speedup 1.0000x reference)
import jax
import jax.numpy as jnp
from jax.experimental import pallas as pl


def kernel(state, Wr1, br1, Wr2, br2, Wh1, bh1, Wh2, bh2, Wo1, bo1, Wo2, bo2, Wc1_root, Wc1_rel, bc1, Wc2_root, Wc2_rel, bc2, Wv1, bv1, Wv2, bv2, Wv3, bv3, dropout):
    raise NotImplementedError("write your pallas kernel here")


# speedup vs baseline: 43.2940x; 43.2940x over previous
import jax
import jax.numpy as jnp
from jax.experimental import pallas as pl

_HUM = 20
_OTH = 11
_SS = 6
_AS = 10
_XD = 256
_BB = 128


def _relu(x):
    return jnp.maximum(x, 0.0)


def _dot(a, b):
    return jax.lax.dot(a, b, precision=jax.lax.Precision.HIGHEST,
                       preferred_element_type=jnp.float32)


def _vn_body(slf, hum, oth,
             wr1, br1, wr2, br2,
             wh1, bh1, wh2, bh2,
             wo1, bo1, wo2, bo2,
             w1d, w1r, bc1,
             w2d, w2r, bc2,
             wv1, bv1, wv2, bv2, wv3, bv3,
             out):
    se = _relu(_dot(_relu(_dot(slf[...], wr1[...]) + br1[...]), wr2[...]) + br2[...])

    h = hum[...].reshape(_HUM * _BB, _AS)
    he = _relu(_dot(_relu(_dot(h, wh1[...]) + bh1[...]), wh2[...]) + bh2[...])
    o = oth[...].reshape(_OTH * _BB, _AS)
    oe = _relu(_dot(_relu(_dot(o, wo1[...]) + bo1[...]), wo2[...]) + bo2[...])

    s0 = se
    for n in range(_HUM):
        s0 = s0 + he[n * _BB:(n + 1) * _BB, :]
    for n in range(_OTH):
        s0 = s0 + oe[n * _BB:(n + 1) * _BB, :]

    t1 = _dot(s0, w1r[...]) + bc1[...]
    h1_0 = _relu(_dot(se, w1d[...]) + t1)
    a_h = _dot(he, w1d[...])
    a_o = _dot(oe, w1d[...])
    s1 = h1_0
    for n in range(_HUM):
        s1 = s1 + _relu(a_h[n * _BB:(n + 1) * _BB, :] + t1)
    for n in range(_OTH):
        s1 = s1 + _relu(a_o[n * _BB:(n + 1) * _BB, :] + t1)

    h2_0 = _relu(_dot(h1_0, w2d[...]) + _dot(s1, w2r[...]) + bc2[...])

    v = _relu(_dot(h2_0, wv1[...]) + bv1[...])
    v = _relu(_dot(v, wv2[...]) + bv2[...])
    out[...] = _dot(v, wv3[...]) + bv3[...]


def kernel(state, Wr1, br1, Wr2, br2, Wh1, bh1, Wh2, bh2, Wo1, bo1, Wo2, bo2,
           Wc1_root, Wc1_rel, bc1, Wc2_root, Wc2_rel, bc2,
           Wv1, bv1, Wv2, bv2, Wv3, bv3, dropout):
    B = state.shape[0]
    slf = state[:, 0, :_SS]
    hum = jnp.transpose(state[:, :_HUM, _SS:], (1, 0, 2))
    oth = jnp.transpose(state[:, _HUM:, _SS:], (1, 0, 2))
    w1d = Wc1_root - Wc1_rel
    w2d = Wc2_root - Wc2_rel

    def r2(b):
        return b.reshape(1, -1)

    def wspec(w):
        n = w.ndim
        return pl.BlockSpec(w.shape, lambda i, _n=n: (0,) * _n)

    weights = [Wr1, r2(br1), Wr2, r2(br2),
               Wh1, r2(bh1), Wh2, r2(bh2),
               Wo1, r2(bo1), Wo2, r2(bo2),
               w1d, Wc1_rel, r2(bc1),
               w2d, Wc2_rel, r2(bc2),
               Wv1, r2(bv1), Wv2, r2(bv2), Wv3, r2(bv3)]

    out = pl.pallas_call(
        _vn_body,
        grid=(B // _BB,),
        in_specs=[
            pl.BlockSpec((_BB, _SS), lambda i: (i, 0)),
            pl.BlockSpec((_HUM, _BB, _AS), lambda i: (0, i, 0)),
            pl.BlockSpec((_OTH, _BB, _AS), lambda i: (0, i, 0)),
        ] + [wspec(w) for w in weights],
        out_specs=pl.BlockSpec((_BB, 1), lambda i: (i, 0)),
        out_shape=jax.ShapeDtypeStruct((B, 1), jnp.float32),
    )(slf, hum, oth, *weights)
    return out


# speedup vs baseline: 53.2668x; 1.2304x over previous
import jax
import jax.numpy as jnp
from jax.experimental import pallas as pl

_HUM = 20
_OTH = 11
_SS = 6
_AS = 10
_XD = 256
_BB = 128


def _relu(x):
    return jnp.maximum(x, 0.0)


def _dot(a, b):
    return jax.lax.dot(a, b, precision=jax.lax.Precision.HIGHEST,
                       preferred_element_type=jnp.float32)


def _bdot(a, b):
    return jax.lax.dot(a, b, preferred_element_type=jnp.float32)


def _split(x):
    hi = x.astype(jnp.bfloat16)
    lo = (x - hi.astype(jnp.float32)).astype(jnp.bfloat16)
    return hi, lo


def _dot3(a, bh, bl):
    ah, al = _split(a)
    return _bdot(ah, bh) + _bdot(al, bh) + _bdot(ah, bl)


def _vn_body(slf, hum, oth,
             wr1, br1, wr2, br2,
             wh1, bh1, wh2h, wh2l, bh2,
             wo1, bo1, wo2h, wo2l, bo2,
             w1dh, w1dl, w1r, bc1,
             w2d, w2r, bc2,
             wv1, bv1, wv2, bv2, wv3, bv3,
             out):
    se = _relu(_dot(_relu(_dot(slf[...], wr1[...]) + br1[...]), wr2[...]) + br2[...])

    h = hum[...].reshape(_HUM * _BB, _AS)
    ph = _relu(_dot(h, wh1[...]) + bh1[...])
    he = _relu(_dot3(ph, wh2h[...], wh2l[...]) + bh2[...])
    o = oth[...].reshape(_OTH * _BB, _AS)
    po = _relu(_dot(o, wo1[...]) + bo1[...])
    oe = _relu(_dot3(po, wo2h[...], wo2l[...]) + bo2[...])

    s0 = se
    for n in range(_HUM):
        s0 = s0 + he[n * _BB:(n + 1) * _BB, :]
    for n in range(_OTH):
        s0 = s0 + oe[n * _BB:(n + 1) * _BB, :]

    t1 = _dot(s0, w1r[...]) + bc1[...]
    h1_0 = _relu(_dot3(se, w1dh[...], w1dl[...]) + t1)
    a_h = _dot3(he, w1dh[...], w1dl[...])
    a_o = _dot3(oe, w1dh[...], w1dl[...])
    s1 = h1_0
    for n in range(_HUM):
        s1 = s1 + _relu(a_h[n * _BB:(n + 1) * _BB, :] + t1)
    for n in range(_OTH):
        s1 = s1 + _relu(a_o[n * _BB:(n + 1) * _BB, :] + t1)

    h2_0 = _relu(_dot(h1_0, w2d[...]) + _dot(s1, w2r[...]) + bc2[...])

    v = _relu(_dot(h2_0, wv1[...]) + bv1[...])
    v = _relu(_dot(v, wv2[...]) + bv2[...])
    out[...] = _dot(v, wv3[...]) + bv3[...]


def kernel(state, Wr1, br1, Wr2, br2, Wh1, bh1, Wh2, bh2, Wo1, bo1, Wo2, bo2,
           Wc1_root, Wc1_rel, bc1, Wc2_root, Wc2_rel, bc2,
           Wv1, bv1, Wv2, bv2, Wv3, bv3, dropout):
    B = state.shape[0]
    slf = state[:, 0, :_SS]
    hum = jnp.transpose(state[:, :_HUM, _SS:], (1, 0, 2))
    oth = jnp.transpose(state[:, _HUM:, _SS:], (1, 0, 2))
    w1d = Wc1_root - Wc1_rel
    w2d = Wc2_root - Wc2_rel
    wh2h, wh2l = _split(Wh2)
    wo2h, wo2l = _split(Wo2)
    w1dh, w1dl = _split(w1d)

    def r2(b):
        return b.reshape(1, -1)

    def wspec(w):
        n = w.ndim
        return pl.BlockSpec(w.shape, lambda i, _n=n: (0,) * _n)

    weights = [Wr1, r2(br1), Wr2, r2(br2),
               Wh1, r2(bh1), wh2h, wh2l, r2(bh2),
               Wo1, r2(bo1), wo2h, wo2l, r2(bo2),
               w1dh, w1dl, Wc1_rel, r2(bc1),
               w2d, Wc2_rel, r2(bc2),
               Wv1, r2(bv1), Wv2, r2(bv2), Wv3, r2(bv3)]

    out = pl.pallas_call(
        _vn_body,
        grid=(B // _BB,),
        in_specs=[
            pl.BlockSpec((_BB, _SS), lambda i: (i, 0)),
            pl.BlockSpec((_HUM, _BB, _AS), lambda i: (0, i, 0)),
            pl.BlockSpec((_OTH, _BB, _AS), lambda i: (0, i, 0)),
        ] + [wspec(w) for w in weights],
        out_specs=pl.BlockSpec((_BB, 1), lambda i: (i, 0)),
        out_shape=jax.ShapeDtypeStruct((B, 1), jnp.float32),
    )(slf, hum, oth, *weights)
    return out
